# Initial kernel scaffold; baseline (speedup 1.0000x reference)
import jax
import jax.numpy as jnp
from jax.experimental import pallas as pl


def kernel(x, edge_index, W1, a1_src, a1_dst, b1, W2, a2_src, a2_dst, b2, W3, a3_src, a3_dst, b3):
    raise NotImplementedError("write your pallas kernel here")


# speedup vs baseline: 23.0203x; 23.0203x over previous
import functools

import jax
import jax.numpy as jnp
from jax import lax
from jax.experimental import pallas as pl
from jax.experimental.pallas import tpu as pltpu
from jax.experimental.pallas import tpu_sc as plsc

N = 10000
D = 128
E = 320000
NPAD = 10240
NC = 2
NS = 16
NW = NC * NS
K = 128
CPW = 81
EW = K * CPW
EP = EW * NW
RPS = NPAD // NS
BLK = 640
GRID = NPAD // BLK

_F32 = jnp.float32


def _tc_first_body(x_ref, w_ref, as_ref, ad_ref, h_ref, aso_ref, ado_ref):
    h = jnp.dot(x_ref[...], w_ref[...], preferred_element_type=_F32)
    h_ref[...] = h
    aso_ref[...] = jnp.sum(h * as_ref[...], axis=1, keepdims=True)
    ado_ref[...] = jnp.sum(h * ad_ref[...], axis=1, keepdims=True)


def _tc_first(x, W, a_s, a_d):
    return pl.pallas_call(
        _tc_first_body,
        grid=(GRID,),
        in_specs=[
            pl.BlockSpec((BLK, D), lambda i: (i, 0)),
            pl.BlockSpec((D, D), lambda i: (0, 0)),
            pl.BlockSpec((1, D), lambda i: (0, 0)),
            pl.BlockSpec((1, D), lambda i: (0, 0)),
        ],
        out_specs=[
            pl.BlockSpec((BLK, D), lambda i: (i, 0)),
            pl.BlockSpec((BLK, 1), lambda i: (i, 0)),
            pl.BlockSpec((BLK, 1), lambda i: (i, 0)),
        ],
        out_shape=[
            jax.ShapeDtypeStruct((NPAD, D), _F32),
            jax.ShapeDtypeStruct((NPAD, 1), _F32),
            jax.ShapeDtypeStruct((NPAD, 1), _F32),
        ],
    )(x, W, a_s, a_d)


def _norm_block(acc_ref, d0_ref, d1_ref, b_ref):
    denom = d0_ref[...] + d1_ref[...] + 1e-16
    return (acc_ref[0] + acc_ref[1]) / denom + b_ref[...]


def _tc_mid_body(acc_ref, d0_ref, d1_ref, b_ref, w_ref, as_ref, ad_ref,
                 h_ref, aso_ref, ado_ref):
    hin = _norm_block(acc_ref, d0_ref, d1_ref, b_ref)
    hin = jnp.maximum(hin, 0.01 * hin)
    h = jnp.dot(hin, w_ref[...], preferred_element_type=_F32)
    h_ref[...] = h
    aso_ref[...] = jnp.sum(h * as_ref[...], axis=1, keepdims=True)
    ado_ref[...] = jnp.sum(h * ad_ref[...], axis=1, keepdims=True)


def _tc_mid(acc, d0, d1, b, W, a_s, a_d):
    return pl.pallas_call(
        _tc_mid_body,
        grid=(GRID,),
        in_specs=[
            pl.BlockSpec((NC, BLK, D), lambda i: (0, i, 0)),
            pl.BlockSpec((BLK, 1), lambda i: (i, 0)),
            pl.BlockSpec((BLK, 1), lambda i: (i, 0)),
            pl.BlockSpec((1, D), lambda i: (0, 0)),
            pl.BlockSpec((D, D), lambda i: (0, 0)),
            pl.BlockSpec((1, D), lambda i: (0, 0)),
            pl.BlockSpec((1, D), lambda i: (0, 0)),
        ],
        out_specs=[
            pl.BlockSpec((BLK, D), lambda i: (i, 0)),
            pl.BlockSpec((BLK, 1), lambda i: (i, 0)),
            pl.BlockSpec((BLK, 1), lambda i: (i, 0)),
        ],
        out_shape=[
            jax.ShapeDtypeStruct((NPAD, D), _F32),
            jax.ShapeDtypeStruct((NPAD, 1), _F32),
            jax.ShapeDtypeStruct((NPAD, 1), _F32),
        ],
    )(acc, d0, d1, b, W, a_s, a_d)


def _tc_last_body(acc_ref, d0_ref, d1_ref, b_ref, out_ref):
    out_ref[...] = _norm_block(acc_ref, d0_ref, d1_ref, b_ref)


def _tc_last(acc, d0, d1, b):
    return pl.pallas_call(
        _tc_last_body,
        grid=(GRID,),
        in_specs=[
            pl.BlockSpec((NC, BLK, D), lambda i: (0, i, 0)),
            pl.BlockSpec((BLK, 1), lambda i: (i, 0)),
            pl.BlockSpec((BLK, 1), lambda i: (i, 0)),
            pl.BlockSpec((1, D), lambda i: (0, 0)),
        ],
        out_specs=pl.BlockSpec((BLK, D), lambda i: (i, 0)),
        out_shape=jax.ShapeDtypeStruct((NPAD, D), _F32),
    )(acc, d0, d1, b)


def _sc_edge_body(h_hbm, as_hbm, ad_hbm, src_hbm, dst_hbm,
                  acc_out, den_out,
                  acc_s, den_s, as_t, ad_t, src_c, dst_c, wbuf, rows, sem):
    cid = lax.axis_index("c")
    sid = lax.axis_index("s")
    wid = sid * NC + cid
    base = sid * RPS

    pltpu.sync_copy(as_hbm, as_t)
    pltpu.sync_copy(ad_hbm, ad_t)

    zv = jnp.zeros((16,), _F32)

    def _zr(r, _):
        for j in range(D // 16):
            rows[r, pl.ds(j * 16, 16)] = zv
        return 0
    lax.fori_loop(0, K, _zr, 0)
    for j in range(K // 16):
        wbuf[pl.ds(j * 16, 16)] = zv

    for i in range(RPS // K):
        pltpu.sync_copy(rows, acc_s.at[pl.ds(base + i * K, K)])
        pltpu.sync_copy(wbuf, den_s.at[pl.ds(base + i * K, K)])
    plsc.subcore_barrier()

    def _chunk(c, _):
        pltpu.sync_copy(src_hbm.at[wid, c], src_c)
        pltpu.sync_copy(dst_hbm.at[wid, c], dst_c)
        gather = pltpu.async_copy(h_hbm.at[src_c.at[0]], rows, sem)
        for j in range(K // 16):
            si = src_c[0, pl.ds(j * 16, 16)]
            di = dst_c[0, pl.ds(j * 16, 16)]
            z = plsc.load_gather(as_t, [si]) + plsc.load_gather(ad_t, [di])
            e = jnp.maximum(z, 0.2 * z)
            wbuf[pl.ds(j * 16, 16)] = jnp.exp(e)
        gather.wait()

        def _scale(k, _):
            wk = plsc.load_gather(wbuf, [jnp.zeros((16,), jnp.int32) + k])
            for j in range(D // 16):
                rows[k, pl.ds(j * 16, 16)] = rows[k, pl.ds(j * 16, 16)] * wk
            return 0
        lax.fori_loop(0, K, _scale, 0)

        pltpu.sync_copy(rows, acc_s.at[dst_c.at[0]], add=True)
        pltpu.sync_copy(wbuf, den_s.at[dst_c.at[0]], add=True)
        return 0

    lax.fori_loop(0, CPW, _chunk, 0)
    plsc.subcore_barrier()

    pltpu.sync_copy(acc_s.at[pl.ds(base, RPS)],
                    acc_out.at[cid, pl.ds(base, RPS)])
    pltpu.sync_copy(den_s.at[pl.ds(base, RPS)],
                    den_out.at[cid, pl.ds(base, RPS)])


_sc_edge = functools.partial(
    pl.kernel,
    _sc_edge_body,
    out_type=[
        jax.ShapeDtypeStruct((NC, NPAD, D), _F32),
        jax.ShapeDtypeStruct((NC, NPAD), _F32),
    ],
    mesh=plsc.VectorSubcoreMesh(core_axis_name="c", subcore_axis_name="s"),
    compiler_params=pltpu.CompilerParams(needs_layout_passes=False),
    scratch_types=[
        pltpu.VMEM_SHARED((NPAD, D), _F32),
        pltpu.VMEM_SHARED((NPAD,), _F32),
        pltpu.VMEM((NPAD,), _F32),
        pltpu.VMEM((NPAD,), _F32),
        pltpu.VMEM((1, K), jnp.int32),
        pltpu.VMEM((1, K), jnp.int32),
        pltpu.VMEM((K,), _F32),
        pltpu.VMEM((K, D), _F32),
        pltpu.SemaphoreType.DMA,
    ],
)()


def kernel(x, edge_index, W1, a1_src, a1_dst, b1, W2, a2_src, a2_dst, b2,
           W3, a3_src, a3_dst, b3):
    ei = edge_index.astype(jnp.int32)
    ar = jnp.arange(N, dtype=jnp.int32)
    pad = jnp.full((EP - E - N,), N, jnp.int32)
    src = jnp.concatenate([ei[0], ar, pad]).reshape(NW, CPW, 1, K)
    dst = jnp.concatenate([ei[1], ar, pad]).reshape(NW, CPW, 1, K)
    xp = jnp.pad(x, ((0, NPAD - N), (0, 0)))

    def layer(h, aso, ado):
        acc, den = _sc_edge(h, aso.reshape(NPAD), ado.reshape(NPAD), src, dst)
        d = den.reshape(NC, NPAD, 1)
        return acc, d[0], d[1]

    r = lambda v: v.reshape(1, D)

    h, aso, ado = _tc_first(xp, W1, r(a1_src), r(a1_dst))
    acc, d0, d1 = layer(h, aso, ado)
    h, aso, ado = _tc_mid(acc, d0, d1, r(b1), W2, r(a2_src), r(a2_dst))
    acc, d0, d1 = layer(h, aso, ado)
    h, aso, ado = _tc_mid(acc, d0, d1, r(b2), W3, r(a3_src), r(a3_dst))
    acc, d0, d1 = layer(h, aso, ado)
    out = _tc_last(acc, d0, d1, r(b3))
    return out[:N]


# speedup vs baseline: 35.0144x; 1.5210x over previous
import functools

import jax
import jax.numpy as jnp
from jax import lax
from jax.experimental import pallas as pl
from jax.experimental.pallas import tpu as pltpu
from jax.experimental.pallas import tpu_sc as plsc

N = 10000
D = 128
E = 320000
NPAD = 10240
NC = 2
NS = 16
NW = NC * NS
KC = 48
CB = 24
NB = 9
EW = KC * CB * NB
EP = EW * NW
RPS = NPAD // NS
BLK = 640
GRID = NPAD // BLK

_F32 = jnp.float32


def _tc_first_body(x_ref, w_ref, as_ref, ad_ref, h_ref, aso_ref, ado_ref):
    h = jnp.dot(x_ref[...], w_ref[...], preferred_element_type=_F32)
    h_ref[...] = h
    aso_ref[...] = jnp.sum(h * as_ref[...], axis=1, keepdims=True)
    ado_ref[...] = jnp.sum(h * ad_ref[...], axis=1, keepdims=True)


def _tc_first(x, W, a_s, a_d):
    return pl.pallas_call(
        _tc_first_body,
        grid=(GRID,),
        in_specs=[
            pl.BlockSpec((BLK, D), lambda i: (i, 0)),
            pl.BlockSpec((D, D), lambda i: (0, 0)),
            pl.BlockSpec((1, D), lambda i: (0, 0)),
            pl.BlockSpec((1, D), lambda i: (0, 0)),
        ],
        out_specs=[
            pl.BlockSpec((BLK, D), lambda i: (i, 0)),
            pl.BlockSpec((BLK, 1), lambda i: (i, 0)),
            pl.BlockSpec((BLK, 1), lambda i: (i, 0)),
        ],
        out_shape=[
            jax.ShapeDtypeStruct((NPAD, D), _F32),
            jax.ShapeDtypeStruct((NPAD, 1), _F32),
            jax.ShapeDtypeStruct((NPAD, 1), _F32),
        ],
    )(x, W, a_s, a_d)


def _norm_block(acc_ref, d0_ref, d1_ref, b_ref):
    denom = d0_ref[...] + d1_ref[...] + 1e-16
    return (acc_ref[0] + acc_ref[1]) / denom + b_ref[...]


def _tc_mid_body(acc_ref, d0_ref, d1_ref, b_ref, w_ref, as_ref, ad_ref,
                 h_ref, aso_ref, ado_ref):
    hin = _norm_block(acc_ref, d0_ref, d1_ref, b_ref)
    hin = jnp.maximum(hin, 0.01 * hin)
    h = jnp.dot(hin, w_ref[...], preferred_element_type=_F32)
    h_ref[...] = h
    aso_ref[...] = jnp.sum(h * as_ref[...], axis=1, keepdims=True)
    ado_ref[...] = jnp.sum(h * ad_ref[...], axis=1, keepdims=True)


def _tc_mid(acc, d0, d1, b, W, a_s, a_d):
    return pl.pallas_call(
        _tc_mid_body,
        grid=(GRID,),
        in_specs=[
            pl.BlockSpec((NC, BLK, D), lambda i: (0, i, 0)),
            pl.BlockSpec((BLK, 1), lambda i: (i, 0)),
            pl.BlockSpec((BLK, 1), lambda i: (i, 0)),
            pl.BlockSpec((1, D), lambda i: (0, 0)),
            pl.BlockSpec((D, D), lambda i: (0, 0)),
            pl.BlockSpec((1, D), lambda i: (0, 0)),
            pl.BlockSpec((1, D), lambda i: (0, 0)),
        ],
        out_specs=[
            pl.BlockSpec((BLK, D), lambda i: (i, 0)),
            pl.BlockSpec((BLK, 1), lambda i: (i, 0)),
            pl.BlockSpec((BLK, 1), lambda i: (i, 0)),
        ],
        out_shape=[
            jax.ShapeDtypeStruct((NPAD, D), _F32),
            jax.ShapeDtypeStruct((NPAD, 1), _F32),
            jax.ShapeDtypeStruct((NPAD, 1), _F32),
        ],
    )(acc, d0, d1, b, W, a_s, a_d)


def _tc_last_body(acc_ref, d0_ref, d1_ref, b_ref, out_ref):
    out_ref[...] = _norm_block(acc_ref, d0_ref, d1_ref, b_ref)


def _tc_last(acc, d0, d1, b):
    return pl.pallas_call(
        _tc_last_body,
        grid=(GRID,),
        in_specs=[
            pl.BlockSpec((NC, BLK, D), lambda i: (0, i, 0)),
            pl.BlockSpec((BLK, 1), lambda i: (i, 0)),
            pl.BlockSpec((BLK, 1), lambda i: (i, 0)),
            pl.BlockSpec((1, D), lambda i: (0, 0)),
        ],
        out_specs=pl.BlockSpec((BLK, D), lambda i: (i, 0)),
        out_shape=jax.ShapeDtypeStruct((NPAD, D), _F32),
    )(acc, d0, d1, b)


def _sc_edge_body(h_hbm, as_hbm, ad_hbm, src_hbm, dst_hbm,
                  acc_out, den_out,
                  acc_s, den_s, as_t, ad_t, srcb, dstb,
                  r0, r1, r2, w0, w1, w2, g0, g1, g2, s0, s1, s2):
    cid = lax.axis_index("c")
    sid = lax.axis_index("s")
    wid = sid * NC + cid
    base = sid * RPS
    rows = (r0, r1, r2)
    wb = (w0, w1, w2)
    gsem = (g0, g1, g2)
    ssem = (s0, s1, s2)

    pltpu.sync_copy(as_hbm, as_t)
    pltpu.sync_copy(ad_hbm, ad_t)

    zv = jnp.zeros((16,), _F32)

    def _zr(r, _):
        for j in range(D // 16):
            r0[r, pl.ds(j * 16, 16)] = zv
        return 0
    lax.fori_loop(0, KC, _zr, 0)
    for j in range(KC // 16):
        w0[pl.ds(j * 16, 16)] = zv

    def _za(i, _):
        pltpu.sync_copy(r0, acc_s.at[pl.ds(base + i * KC, KC)])
        pltpu.sync_copy(w0, den_s.at[pl.ds(base + i * KC, KC)])
        return 0
    lax.fori_loop(0, RPS // 16 // 3, _za, 0)
    pltpu.sync_copy(r0.at[pl.ds(0, 16)],
                    acc_s.at[pl.ds(base + 13 * KC, 16)])
    pltpu.sync_copy(w0.at[pl.ds(0, 16)],
                    den_s.at[pl.ds(base + 13 * KC, 16)])
    plsc.subcore_barrier()

    def _drain(p):
        pltpu.make_async_copy(rows[p], acc_s.at[dstb.at[0]], ssem[p]).wait()
        pltpu.make_async_copy(wb[p], den_s.at[dstb.at[0]], ssem[p]).wait()

    def _proc(p, q):
        for j in range(KC // 16):
            si = srcb[q, pl.ds(j * 16, 16)]
            di = dstb[q, pl.ds(j * 16, 16)]
            z = plsc.load_gather(as_t, [si]) + plsc.load_gather(ad_t, [di])
            e = jnp.maximum(z, 0.2 * z)
            wb[p][pl.ds(j * 16, 16)] = jnp.exp(e)

        pn = (p + 1) % 3

        @pl.when(q >= 2)
        def _():
            _drain(pn)

        @pl.when(q + 1 < CB)
        def _():
            pltpu.async_copy(h_hbm.at[srcb.at[q + 1]], rows[pn], gsem[pn])

        pltpu.make_async_copy(h_hbm.at[srcb.at[q]], rows[p], gsem[p]).wait()

        def _scale(k, _):
            wk = plsc.load_gather(wb[p], [jnp.zeros((16,), jnp.int32) + k])
            for j in range(D // 16):
                rows[p][k, pl.ds(j * 16, 16)] = rows[p][k, pl.ds(j * 16, 16)] * wk
            return 0
        lax.fori_loop(0, KC, _scale, 0)

        pltpu.async_copy(rows[p], acc_s.at[dstb.at[q]], ssem[p], add=True)
        pltpu.async_copy(wb[p], den_s.at[dstb.at[q]], ssem[p], add=True)

    def _block(blk, _):
        @pl.when(blk >= 1)
        def _():
            _drain(1)
            _drain(2)
        pltpu.sync_copy(src_hbm.at[wid, blk], srcb)
        pltpu.sync_copy(dst_hbm.at[wid, blk], dstb)
        pltpu.async_copy(h_hbm.at[srcb.at[0]], rows[0], gsem[0])

        def _tri(i, _):
            _proc(0, 3 * i)
            _proc(1, 3 * i + 1)
            _proc(2, 3 * i + 2)
            return 0
        lax.fori_loop(0, CB // 3, _tri, 0)
        return 0

    lax.fori_loop(0, NB, _block, 0)
    _drain(1)
    _drain(2)
    plsc.subcore_barrier()

    pltpu.sync_copy(acc_s.at[pl.ds(base, RPS)],
                    acc_out.at[cid, pl.ds(base, RPS)])
    pltpu.sync_copy(den_s.at[pl.ds(base, RPS)],
                    den_out.at[cid, pl.ds(base, RPS)])


_sc_edge = functools.partial(
    pl.kernel,
    _sc_edge_body,
    out_type=[
        jax.ShapeDtypeStruct((NC, NPAD, D), _F32),
        jax.ShapeDtypeStruct((NC, NPAD), _F32),
    ],
    mesh=plsc.VectorSubcoreMesh(core_axis_name="c", subcore_axis_name="s"),
    compiler_params=pltpu.CompilerParams(needs_layout_passes=False),
    scratch_types=[
        pltpu.VMEM_SHARED((NPAD, D), _F32),
        pltpu.VMEM_SHARED((NPAD,), _F32),
        pltpu.VMEM((NPAD,), _F32),
        pltpu.VMEM((NPAD,), _F32),
        pltpu.VMEM((CB, KC), jnp.int32),
        pltpu.VMEM((CB, KC), jnp.int32),
        pltpu.VMEM((KC, D), _F32),
        pltpu.VMEM((KC, D), _F32),
        pltpu.VMEM((KC, D), _F32),
        pltpu.VMEM((KC,), _F32),
        pltpu.VMEM((KC,), _F32),
        pltpu.VMEM((KC,), _F32),
        pltpu.SemaphoreType.DMA,
        pltpu.SemaphoreType.DMA,
        pltpu.SemaphoreType.DMA,
        pltpu.SemaphoreType.DMA,
        pltpu.SemaphoreType.DMA,
        pltpu.SemaphoreType.DMA,
    ],
)()


def kernel(x, edge_index, W1, a1_src, a1_dst, b1, W2, a2_src, a2_dst, b2,
           W3, a3_src, a3_dst, b3):
    ei = edge_index.astype(jnp.int32)
    ar = jnp.arange(N, dtype=jnp.int32)
    pad = jnp.full((EP - E - N,), N, jnp.int32)
    src = jnp.concatenate([ei[0], ar, pad]).reshape(NW, NB, CB, KC)
    dst = jnp.concatenate([ei[1], ar, pad]).reshape(NW, NB, CB, KC)
    xp = jnp.pad(x, ((0, NPAD - N), (0, 0)))

    def layer(h, aso, ado):
        acc, den = _sc_edge(h, aso.reshape(NPAD), ado.reshape(NPAD), src, dst)
        d = den.reshape(NC, NPAD, 1)
        return acc, d[0], d[1]

    r = lambda v: v.reshape(1, D)

    h, aso, ado = _tc_first(xp, W1, r(a1_src), r(a1_dst))
    acc, d0, d1 = layer(h, aso, ado)
    h, aso, ado = _tc_mid(acc, d0, d1, r(b1), W2, r(a2_src), r(a2_dst))
    acc, d0, d1 = layer(h, aso, ado)
    h, aso, ado = _tc_mid(acc, d0, d1, r(b2), W3, r(a3_src), r(a3_dst))
    acc, d0, d1 = layer(h, aso, ado)
    out = _tc_last(acc, d0, d1, r(b3))
    return out[:N]


# speedup vs baseline: 36.5663x; 1.0443x over previous
import functools

import jax
import jax.numpy as jnp
from jax import lax
from jax.experimental import pallas as pl
from jax.experimental.pallas import tpu as pltpu
from jax.experimental.pallas import tpu_sc as plsc

N = 10000
D = 128
E = 320000
NPAD = 10240
NC = 2
NS = 16
NW = NC * NS
KC = 48
CB = 24
NB = 9
EW = KC * CB * NB
EP = EW * NW
RPS = NPAD // NS
BLK = 640
GRID = NPAD // BLK

_F32 = jnp.float32


def _tc_first_body(x_ref, w_ref, as_ref, ad_ref, h_ref, aso_ref, ado_ref):
    h = jnp.dot(x_ref[...], w_ref[...], preferred_element_type=_F32)
    h_ref[...] = h
    aso_ref[...] = jnp.sum(h * as_ref[...], axis=1, keepdims=True)
    ado_ref[...] = jnp.sum(h * ad_ref[...], axis=1, keepdims=True)


def _tc_first(x, W, a_s, a_d):
    return pl.pallas_call(
        _tc_first_body,
        grid=(GRID,),
        in_specs=[
            pl.BlockSpec((BLK, D), lambda i: (i, 0)),
            pl.BlockSpec((D, D), lambda i: (0, 0)),
            pl.BlockSpec((1, D), lambda i: (0, 0)),
            pl.BlockSpec((1, D), lambda i: (0, 0)),
        ],
        out_specs=[
            pl.BlockSpec((BLK, D), lambda i: (i, 0)),
            pl.BlockSpec((BLK, 1), lambda i: (i, 0)),
            pl.BlockSpec((BLK, 1), lambda i: (i, 0)),
        ],
        out_shape=[
            jax.ShapeDtypeStruct((NPAD, D), _F32),
            jax.ShapeDtypeStruct((NPAD, 1), _F32),
            jax.ShapeDtypeStruct((NPAD, 1), _F32),
        ],
    )(x, W, a_s, a_d)


def _norm_block(acc_ref, d0_ref, d1_ref, b_ref):
    denom = d0_ref[...] + d1_ref[...] + 1e-16
    return (acc_ref[0] + acc_ref[1]) / denom + b_ref[...]


def _tc_mid_body(acc_ref, d0_ref, d1_ref, b_ref, w_ref, as_ref, ad_ref,
                 h_ref, aso_ref, ado_ref):
    hin = _norm_block(acc_ref, d0_ref, d1_ref, b_ref)
    hin = jnp.maximum(hin, 0.01 * hin)
    h = jnp.dot(hin, w_ref[...], preferred_element_type=_F32)
    h_ref[...] = h
    aso_ref[...] = jnp.sum(h * as_ref[...], axis=1, keepdims=True)
    ado_ref[...] = jnp.sum(h * ad_ref[...], axis=1, keepdims=True)


def _tc_mid(acc, d0, d1, b, W, a_s, a_d):
    return pl.pallas_call(
        _tc_mid_body,
        grid=(GRID,),
        in_specs=[
            pl.BlockSpec((NC, BLK, D), lambda i: (0, i, 0)),
            pl.BlockSpec((BLK, 1), lambda i: (i, 0)),
            pl.BlockSpec((BLK, 1), lambda i: (i, 0)),
            pl.BlockSpec((1, D), lambda i: (0, 0)),
            pl.BlockSpec((D, D), lambda i: (0, 0)),
            pl.BlockSpec((1, D), lambda i: (0, 0)),
            pl.BlockSpec((1, D), lambda i: (0, 0)),
        ],
        out_specs=[
            pl.BlockSpec((BLK, D), lambda i: (i, 0)),
            pl.BlockSpec((BLK, 1), lambda i: (i, 0)),
            pl.BlockSpec((BLK, 1), lambda i: (i, 0)),
        ],
        out_shape=[
            jax.ShapeDtypeStruct((NPAD, D), _F32),
            jax.ShapeDtypeStruct((NPAD, 1), _F32),
            jax.ShapeDtypeStruct((NPAD, 1), _F32),
        ],
    )(acc, d0, d1, b, W, a_s, a_d)


def _tc_last_body(acc_ref, d0_ref, d1_ref, b_ref, out_ref):
    out_ref[...] = _norm_block(acc_ref, d0_ref, d1_ref, b_ref)


def _tc_last(acc, d0, d1, b):
    return pl.pallas_call(
        _tc_last_body,
        grid=(GRID,),
        in_specs=[
            pl.BlockSpec((NC, BLK, D), lambda i: (0, i, 0)),
            pl.BlockSpec((BLK, 1), lambda i: (i, 0)),
            pl.BlockSpec((BLK, 1), lambda i: (i, 0)),
            pl.BlockSpec((1, D), lambda i: (0, 0)),
        ],
        out_specs=pl.BlockSpec((BLK, D), lambda i: (i, 0)),
        out_shape=jax.ShapeDtypeStruct((NPAD, D), _F32),
    )(acc, d0, d1, b)


def _sc_edge_body(h_hbm, as_hbm, ad_hbm, src_hbm, dst_hbm,
                  acc_out, den_out,
                  acc_s, den_s, as_t, ad_t, srcb, dstb,
                  r0, r1, r2, w0, w1, w2, g0, g1, g2, s0, s1, s2):
    cid = lax.axis_index("c")
    sid = lax.axis_index("s")
    wid = sid * NC + cid
    base = sid * RPS
    rows = (r0, r1, r2)
    wb = (w0, w1, w2)
    gsem = (g0, g1, g2)
    ssem = (s0, s1, s2)

    pltpu.sync_copy(as_hbm, as_t)
    pltpu.sync_copy(ad_hbm, ad_t)

    zv = jnp.zeros((16,), _F32)

    def _zr(r, _):
        for j in range(D // 16):
            r0[r, pl.ds(j * 16, 16)] = zv
        return 0
    lax.fori_loop(0, KC, _zr, 0)
    for j in range(KC // 16):
        w0[pl.ds(j * 16, 16)] = zv

    def _za(i, _):
        pltpu.sync_copy(r0, acc_s.at[pl.ds(base + i * KC, KC)])
        pltpu.sync_copy(w0, den_s.at[pl.ds(base + i * KC, KC)])
        return 0
    lax.fori_loop(0, RPS // 16 // 3, _za, 0)
    pltpu.sync_copy(r0.at[pl.ds(0, 16)],
                    acc_s.at[pl.ds(base + 13 * KC, 16)])
    pltpu.sync_copy(w0.at[pl.ds(0, 16)],
                    den_s.at[pl.ds(base + 13 * KC, 16)])
    plsc.subcore_barrier()

    def _drain(p):
        pltpu.make_async_copy(rows[p], acc_s.at[dstb.at[0]], ssem[p]).wait()
        pltpu.make_async_copy(wb[p], den_s.at[dstb.at[0]], ssem[p]).wait()

    def _proc(p, q):
        for j in range(KC // 16):
            si = srcb[q, pl.ds(j * 16, 16)]
            di = dstb[q, pl.ds(j * 16, 16)]
            z = plsc.load_gather(as_t, [si]) + plsc.load_gather(ad_t, [di])
            e = jnp.maximum(z, 0.2 * z)
            wb[p][pl.ds(j * 16, 16)] = jnp.exp(e)

        pn = (p + 1) % 3

        @pl.when(q >= 2)
        def _():
            _drain(pn)

        @pl.when(q + 1 < CB)
        def _():
            pltpu.async_copy(h_hbm.at[srcb.at[q + 1]], rows[pn], gsem[pn])

        pltpu.make_async_copy(h_hbm.at[srcb.at[q]], rows[p], gsem[p]).wait()

        def _scale(k2, _):
            k = k2 * 2
            b0 = jnp.zeros((16,), jnp.int32) + k
            wk0 = plsc.load_gather(wb[p], [b0])
            wk1 = plsc.load_gather(wb[p], [b0 + 1])
            for j in range(D // 16):
                rows[p][k, pl.ds(j * 16, 16)] = rows[p][k, pl.ds(j * 16, 16)] * wk0
            for j in range(D // 16):
                rows[p][k + 1, pl.ds(j * 16, 16)] = rows[p][k + 1, pl.ds(j * 16, 16)] * wk1
            return 0
        lax.fori_loop(0, KC // 2, _scale, 0)

        pltpu.async_copy(rows[p], acc_s.at[dstb.at[q]], ssem[p], add=True)
        pltpu.async_copy(wb[p], den_s.at[dstb.at[q]], ssem[p], add=True)

    def _block(blk, _):
        @pl.when(blk >= 1)
        def _():
            _drain(1)
            _drain(2)
        pltpu.sync_copy(src_hbm.at[wid, blk], srcb)
        pltpu.sync_copy(dst_hbm.at[wid, blk], dstb)
        pltpu.async_copy(h_hbm.at[srcb.at[0]], rows[0], gsem[0])

        def _tri(i, _):
            _proc(0, 3 * i)
            _proc(1, 3 * i + 1)
            _proc(2, 3 * i + 2)
            return 0
        lax.fori_loop(0, CB // 3, _tri, 0)
        return 0

    lax.fori_loop(0, NB, _block, 0)
    _drain(1)
    _drain(2)
    plsc.subcore_barrier()

    pltpu.sync_copy(acc_s.at[pl.ds(base, RPS)],
                    acc_out.at[cid, pl.ds(base, RPS)])
    pltpu.sync_copy(den_s.at[pl.ds(base, RPS)],
                    den_out.at[cid, pl.ds(base, RPS)])


_sc_edge = functools.partial(
    pl.kernel,
    _sc_edge_body,
    out_type=[
        jax.ShapeDtypeStruct((NC, NPAD, D), _F32),
        jax.ShapeDtypeStruct((NC, NPAD), _F32),
    ],
    mesh=plsc.VectorSubcoreMesh(core_axis_name="c", subcore_axis_name="s"),
    compiler_params=pltpu.CompilerParams(needs_layout_passes=False),
    scratch_types=[
        pltpu.VMEM_SHARED((NPAD, D), _F32),
        pltpu.VMEM_SHARED((NPAD,), _F32),
        pltpu.VMEM((NPAD,), _F32),
        pltpu.VMEM((NPAD,), _F32),
        pltpu.VMEM((CB, KC), jnp.int32),
        pltpu.VMEM((CB, KC), jnp.int32),
        pltpu.VMEM((KC, D), _F32),
        pltpu.VMEM((KC, D), _F32),
        pltpu.VMEM((KC, D), _F32),
        pltpu.VMEM((KC,), _F32),
        pltpu.VMEM((KC,), _F32),
        pltpu.VMEM((KC,), _F32),
        pltpu.SemaphoreType.DMA,
        pltpu.SemaphoreType.DMA,
        pltpu.SemaphoreType.DMA,
        pltpu.SemaphoreType.DMA,
        pltpu.SemaphoreType.DMA,
        pltpu.SemaphoreType.DMA,
    ],
)()


def kernel(x, edge_index, W1, a1_src, a1_dst, b1, W2, a2_src, a2_dst, b2,
           W3, a3_src, a3_dst, b3):
    ei = edge_index.astype(jnp.int32)
    ar = jnp.arange(N, dtype=jnp.int32)
    pad = jnp.full((EP - E - N,), N, jnp.int32)
    src = jnp.concatenate([ei[0], ar, pad]).reshape(NW, NB, CB, KC)
    dst = jnp.concatenate([ei[1], ar, pad]).reshape(NW, NB, CB, KC)
    xp = jnp.pad(x, ((0, NPAD - N), (0, 0)))

    def layer(h, aso, ado):
        acc, den = _sc_edge(h, aso.reshape(NPAD), ado.reshape(NPAD), src, dst)
        d = den.reshape(NC, NPAD, 1)
        return acc, d[0], d[1]

    r = lambda v: v.reshape(1, D)

    h, aso, ado = _tc_first(xp, W1, r(a1_src), r(a1_dst))
    acc, d0, d1 = layer(h, aso, ado)
    h, aso, ado = _tc_mid(acc, d0, d1, r(b1), W2, r(a2_src), r(a2_dst))
    acc, d0, d1 = layer(h, aso, ado)
    h, aso, ado = _tc_mid(acc, d0, d1, r(b2), W3, r(a3_src), r(a3_dst))
    acc, d0, d1 = layer(h, aso, ado)
    out = _tc_last(acc, d0, d1, r(b3))
    return out[:N]
